# baseline (speedup 1.0000x reference)
import functools
import jax
import jax.numpy as jnp
from jax import lax
from jax.experimental import pallas as pl
from jax.experimental.pallas import tpu as pltpu
from jax.experimental.pallas import tpu_sc as plsc

N = 10000
D = 128
E = 320000
NC = 2
NS = 16
NW = NC * NS
EPW = E // NW
CHUNK = 80
NCHUNK = 128
BLK_CH = 8
NBLK = NCHUNK // BLK_CH
EPT = NCHUNK * CHUNK
GROUPS = CHUNK // 16
RPT = 624
TAIL = N - NS * RPT


def _bcast_lane(v16, lane):
    idx = jnp.full((16, 1), lane, dtype=jnp.int32)
    return lax.gather(
        v16, idx,
        dimension_numbers=lax.GatherDimensionNumbers(
            offset_dims=(), collapsed_slice_dims=(0,), start_index_map=(0,)),
        slice_sizes=(1,),
        mode=lax.GatherScatterMode.PROMISE_IN_BOUNDS)


@functools.partial(
    pl.kernel,
    out_type=jax.ShapeDtypeStruct((2 * N, D), jnp.float32),
    mesh=plsc.VectorSubcoreMesh(core_axis_name="c", subcore_axis_name="s",
                                num_cores=NC, num_subcores=NS),
    scratch_types=[
        pltpu.VMEM((NCHUNK, CHUNK), jnp.int32),
        pltpu.VMEM((2, BLK_CH, CHUNK), jnp.int32),
        pltpu.VMEM((2, BLK_CH * CHUNK), jnp.float32),
        pltpu.VMEM((CHUNK, D), jnp.float32),
        pltpu.VMEM((CHUNK, D), jnp.float32),
        pltpu.VMEM_SHARED((N, D), jnp.float32),
        pltpu.SemaphoreType.DMA,
        pltpu.SemaphoreType.DMA,
        pltpu.SemaphoreType.DMA,
    ],
)
def _sc_spmm(g_hbm, col_hbm, row_hbm, val_hbm, zero_hbm, p_hbm,
             col_v, row_sl, val_sl, rows_a, rows_b, acc_sh,
             sem_a, sem_b, sem_i):
    c_ax = lax.axis_index("c")
    s_ax = lax.axis_index("s")
    wid = s_ax * NC + c_ax
    cbase = wid * NCHUNK
    ebase = wid * EPT

    pltpu.sync_copy(zero_hbm.at[pl.ds(0, RPT)],
                    acc_sh.at[pl.ds(s_ax * RPT, RPT)])

    @pl.when(s_ax == 0)
    def _():
        pltpu.sync_copy(zero_hbm.at[pl.ds(0, TAIL)],
                        acc_sh.at[pl.ds(NS * RPT, TAIL)])

    pltpu.sync_copy(col_hbm.at[wid], col_v)
    plsc.subcore_barrier()

    def stage_block(b, p):
        rows = pl.ds(cbase + b * BLK_CH, BLK_CH)
        flat = pl.ds(ebase + b * BLK_CH * CHUNK, BLK_CH * CHUNK)
        pltpu.async_copy(row_hbm.at[rows], row_sl.at[p], sem_i)
        pltpu.async_copy(val_hbm.at[flat], val_sl.at[p], sem_i)

    def wait_block(b, p):
        rows = pl.ds(cbase + b * BLK_CH, BLK_CH)
        flat = pl.ds(ebase + b * BLK_CH * CHUNK, BLK_CH * CHUNK)
        pltpu.make_async_copy(row_hbm.at[rows], row_sl.at[p], sem_i).wait()
        pltpu.make_async_copy(val_hbm.at[flat], val_sl.at[p], sem_i).wait()

    def gather(j, buf, sem):
        pltpu.async_copy(g_hbm.at[col_v.at[j]], buf, sem)

    def wait_gather(j, buf, sem):
        pltpu.make_async_copy(g_hbm.at[col_v.at[j]], buf, sem).wait()

    def process(jj, buf, p):
        def group(g, carry):
            v16 = val_sl[p, pl.ds(jj * CHUNK + g * 16, 16)]
            for i in range(16):
                e = g * 16 + i
                sc = _bcast_lane(v16, i)
                for d in range(D // 16):
                    sl = pl.ds(d * 16, 16)
                    buf[e, sl] = buf[e, sl] * sc
            return carry

        lax.fori_loop(0, GROUPS, group, 0)
        pltpu.sync_copy(buf, acc_sh.at[row_sl.at[p, jj]], add=True)

    stage_block(0, 0)
    stage_block(1, 1)
    gather(0, rows_a, sem_a)
    gather(1, rows_b, sem_b)

    def outer(k, carry):
        for pslot in range(2):
            b = 2 * k + pslot
            wait_block(b, pslot)

            def inner(m, carry2):
                c0 = b * BLK_CH + 2 * m
                wait_gather(c0, rows_a, sem_a)
                process(2 * m, rows_a, pslot)

                @pl.when(c0 + 2 < NCHUNK)
                def _():
                    gather(c0 + 2, rows_a, sem_a)

                wait_gather(c0 + 1, rows_b, sem_b)
                process(2 * m + 1, rows_b, pslot)

                @pl.when(c0 + 3 < NCHUNK)
                def _():
                    gather(c0 + 3, rows_b, sem_b)

                return carry2

            lax.fori_loop(0, BLK_CH // 2, inner, 0)

            @pl.when(b + 2 < NBLK)
            def _():
                stage_block(b + 2, pslot)
        return carry

    lax.fori_loop(0, NBLK // 2, outer, 0)

    plsc.subcore_barrier()
    pltpu.sync_copy(acc_sh.at[pl.ds(s_ax * RPT, RPT)],
                    p_hbm.at[pl.ds(c_ax * N + s_ax * RPT, RPT)])

    @pl.when(s_ax == 0)
    def _():
        pltpu.sync_copy(acc_sh.at[pl.ds(NS * RPT, TAIL)],
                        p_hbm.at[pl.ds(c_ax * N + NS * RPT, TAIL)])


_BLK = 2000


def _mm_a_body(h_ref, w_ref, o_ref):
    o_ref[...] = jnp.dot(h_ref[...], w_ref[...],
                         preferred_element_type=jnp.float32)


def _mm_b_body(h_ref, w_ref, b_ref, p0_ref, p1_ref, o_ref):
    o_ref[...] = (jnp.dot(h_ref[...], w_ref[...],
                          preferred_element_type=jnp.float32)
                  + b_ref[...] + p0_ref[...] + p1_ref[...])


def kernel(H, A_indices, A_values, W, b):
    pad = jnp.zeros((NW, EPT - EPW), jnp.int32)
    col = jnp.concatenate(
        [A_indices[1].astype(jnp.int32).reshape(NW, EPW), pad],
        axis=1).reshape(NW, NCHUNK, CHUNK)
    row = jnp.concatenate(
        [A_indices[0].astype(jnp.int32).reshape(NW, EPW), pad],
        axis=1).reshape(NW * NCHUNK, CHUNK)
    val = jnp.concatenate(
        [A_values.reshape(NW, EPW), pad.astype(jnp.float32)],
        axis=1).reshape(NW * EPT)
    w1t = W[:, :D].T
    w2t = W[:, D:].T
    zeros = jnp.zeros((RPT, D), jnp.float32)
    b2 = b.reshape(1, D)

    G = pl.pallas_call(
        _mm_a_body,
        grid=(N // _BLK,),
        in_specs=[
            pl.BlockSpec((_BLK, D), lambda i: (i, 0)),
            pl.BlockSpec((D, D), lambda i: (0, 0)),
        ],
        out_specs=pl.BlockSpec((_BLK, D), lambda i: (i, 0)),
        out_shape=jax.ShapeDtypeStruct((N, D), jnp.float32),
    )(H, w2t)

    P = _sc_spmm(G, col, row, val, zeros)

    Y = pl.pallas_call(
        _mm_b_body,
        grid=(N // _BLK,),
        in_specs=[
            pl.BlockSpec((_BLK, D), lambda i: (i, 0)),
            pl.BlockSpec((D, D), lambda i: (0, 0)),
            pl.BlockSpec((1, D), lambda i: (0, 0)),
            pl.BlockSpec((_BLK, D), lambda i: (i, 0)),
            pl.BlockSpec((_BLK, D), lambda i: (i + N // _BLK, 0)),
        ],
        out_specs=pl.BlockSpec((_BLK, D), lambda i: (i, 0)),
        out_shape=jax.ShapeDtypeStruct((N, D), jnp.float32),
    )(H, w1t, b2, P, P)

    return Y
